# Initial kernel scaffold; baseline (speedup 1.0000x reference)
import jax
import jax.numpy as jnp
from jax.experimental import pallas as pl


def kernel(x, masked_token, shuffle_indices):
    raise NotImplementedError("write your pallas kernel here")


# speedup vs baseline: 2.4712x; 2.4712x over previous
import jax
import jax.numpy as jnp
from jax import lax
from jax.experimental import pallas as pl
from jax.experimental.pallas import tpu as pltpu
from jax.experimental.pallas import tpu_sc as plsc

N = 64
HH = 512
ROWW = 1536
G = 32
NP = G * G
NUM_MASKED = 768
PR = 16
PW = 48
NC, NS = 2, 16
NW = NC * NS


def _sc_body(x_hbm, idx_hbm, tok_hbm, out_hbm,
             tok_v, w0, w1, seg_v, idx_v, mask_v, dirty0_sm, dirty1_sm,
             gsem, osem0, osem1):
    wid = lax.axis_index("s") * NC + lax.axis_index("c")

    pltpu.sync_copy(tok_hbm, tok_v)
    pltpu.sync_copy(tok_hbm, w0)
    pltpu.sync_copy(tok_hbm, w1)
    dirty0_sm[0] = 0
    dirty1_sm[0] = 0

    zeros16 = jnp.zeros((16,), jnp.int32)
    ones16 = jnp.ones((16,), jnp.int32)

    def do_chunk(s, b, gh, w, dirty_sm, osem):
        first = jnp.logical_and(gh < 2, s == 0)
        @pl.when(jnp.logical_not(first))
        def _():
            pltpu.make_async_copy(w, out_hbm.at[b, pl.ds(0, PR), :], osem).wait()
        nd = dirty_sm[0]

        def rbody(j, _):
            c0 = dirty_sm[1 + j] * PW
            for r in range(PR):
                for q in range(3):
                    w[r, pl.ds(c0 + q * 16, 16)] = tok_v[r, pl.ds(c0 + q * 16, 16)]
            return 0
        lax.fori_loop(0, nd, rbody, 0)
        dirty_sm[0] = 0
        r0 = gh * PR

        def gbody(gw, _):
            mv = mask_v[pl.ds(gh * G + gw, 16)]

            @pl.when(mv[0] > 0)
            def _():
                cnt = dirty_sm[0]
                c0 = gw * PW
                pltpu.make_async_copy(
                    x_hbm.at[b, pl.ds(r0, PR), pl.ds(c0, PW)],
                    w.at[:, pl.ds(c0, PW)],
                    gsem).start()
                dirty_sm[1 + cnt] = gw
                dirty_sm[0] = cnt + 1
            return 0
        lax.fori_loop(0, G, gbody, 0)

        def dbody(j, _):
            pltpu.make_async_copy(
                x_hbm.at[0, pl.ds(0, PR), pl.ds(0, PW)], seg_v, gsem).wait()
            return 0
        lax.fori_loop(0, dirty_sm[0], dbody, 0)
        pltpu.make_async_copy(w, out_hbm.at[b, pl.ds(r0, PR), :], osem).start()

    def process_sample(s):
        b = 2 * wid + s
        pltpu.sync_copy(idx_hbm.at[b], idx_v)

        def zbody(j, _):
            mask_v[pl.ds(j * 16, 16)] = zeros16
            return 0
        lax.fori_loop(0, NP // 16, zbody, 0)

        def sbody(j, _):
            iv = idx_v[pl.ds(NUM_MASKED + j * 16, 16)]
            plsc.store_scatter(mask_v, [iv], ones16)
            return 0
        lax.fori_loop(0, (NP - NUM_MASKED) // 16, sbody, 0)

        def pair(t, _):
            do_chunk(s, b, 2 * t, w0, dirty0_sm, osem0)
            do_chunk(s, b, 2 * t + 1, w1, dirty1_sm, osem1)
            return 0
        lax.fori_loop(0, G // 2, pair, 0)

    process_sample(0)
    process_sample(1)
    pltpu.make_async_copy(w0, out_hbm.at[0, pl.ds(0, PR), :], osem0).wait()
    pltpu.make_async_copy(w1, out_hbm.at[0, pl.ds(0, PR), :], osem1).wait()


def kernel(x, masked_token, shuffle_indices):
    xr = x.reshape(N, HH, ROWW)
    idx = shuffle_indices.astype(jnp.int32)
    tok_t = jnp.tile(masked_token.reshape(PR, PW), (1, G))
    mesh = plsc.VectorSubcoreMesh(core_axis_name="c", subcore_axis_name="s",
                                  num_cores=NC, num_subcores=NS)
    f = pl.kernel(
        _sc_body,
        out_type=jax.ShapeDtypeStruct((N, HH, ROWW), jnp.float32),
        mesh=mesh,
        compiler_params=pltpu.CompilerParams(use_tc_tiling_on_sc=False,
                                             needs_layout_passes=False),
        scratch_types=[
            pltpu.VMEM((PR, ROWW), jnp.float32),
            pltpu.VMEM((PR, ROWW), jnp.float32),
            pltpu.VMEM((PR, ROWW), jnp.float32),
            pltpu.VMEM((PR, PW), jnp.float32),
            pltpu.VMEM((NP,), jnp.int32),
            pltpu.VMEM((NP + 16,), jnp.int32),
            pltpu.SMEM((G + 1,), jnp.int32),
            pltpu.SMEM((G + 1,), jnp.int32),
            pltpu.SemaphoreType.DMA,
            pltpu.SemaphoreType.DMA,
            pltpu.SemaphoreType.DMA,
        ],
    )
    out = f(xr, idx, tok_t)
    return out.reshape(N, HH, HH, 3)
